# baseline (speedup 1.0000x reference)
import functools

import jax
import jax.numpy as jnp
from jax import lax
from jax.experimental import pallas as pl
from jax.experimental.pallas import tpu as pltpu
from jax.experimental.pallas import tpu_sc as plsc

N = 10000
NP = 10240
E = 320000
HID = 128
NCORES = 2
NSUB = 16
NW = NCORES * NSUB
CHUNK = 128
NCHUNK = 79
EPW = NCHUNK * CHUNK
EPAD = EPW * NW
SL = NP // NSUB
DUMMY = N + 100

_mesh = plsc.VectorSubcoreMesh(core_axis_name="c", subcore_axis_name="s",
                               num_cores=NCORES, num_subcores=NSUB)


@functools.partial(
    pl.kernel,
    out_type=jax.ShapeDtypeStruct((NCORES, NP), jnp.float32),
    mesh=_mesh,
    scratch_types=[
        pltpu.VMEM((NCHUNK, CHUNK), jnp.int32),
        pltpu.VMEM((CHUNK,), jnp.float32),
        pltpu.VMEM((SL,), jnp.float32),
        pltpu.VMEM_SHARED((NP,), jnp.float32),
        pltpu.SemaphoreType.DMA,
    ],
)
def _sc_degree(dst_hbm, out_hbm, dst_v, ones_v, zer_v, acc_sh, sem):
    c = lax.axis_index("c")
    s = lax.axis_index("s")
    wid = c * NSUB + s
    pltpu.sync_copy(dst_hbm.at[wid], dst_v)
    for i in range(CHUNK // 16):
        ones_v[pl.ds(i * 16, 16)] = jnp.full((16,), 1.0, jnp.float32)
    for i in range(SL // 16):
        zer_v[pl.ds(i * 16, 16)] = jnp.zeros((16,), jnp.float32)
    pltpu.sync_copy(zer_v, acc_sh.at[pl.ds(s * SL, SL)])
    plsc.subcore_barrier()

    def body(j, carry):
        pltpu.sync_copy(ones_v, acc_sh.at[dst_v.at[j]], add=True)
        return carry

    lax.fori_loop(0, NCHUNK, body, 0)
    plsc.subcore_barrier()
    pltpu.sync_copy(acc_sh.at[pl.ds(s * SL, SL)], out_hbm.at[c, pl.ds(s * SL, SL)])


@functools.partial(
    pl.kernel,
    out_type=jax.ShapeDtypeStruct((NCORES, NP, HID), jnp.float32),
    mesh=_mesh,
    scratch_types=[
        pltpu.VMEM((2, CHUNK), jnp.int32),
        pltpu.VMEM((NCHUNK, CHUNK), jnp.int32),
        pltpu.VMEM((2, CHUNK, HID), jnp.float32),
        pltpu.VMEM_SHARED((NP, HID), jnp.float32),
        pltpu.SemaphoreType.DMA,
    ],
)
def _sc_layer(src_hbm, dst_hbm, tbl_hbm, zer_hbm, out_hbm,
              src_v, dst_v, rows_v, acc_sh, sem):
    c = lax.axis_index("c")
    s = lax.axis_index("s")
    wid = c * NSUB + s
    pltpu.sync_copy(dst_hbm.at[wid], dst_v)
    pltpu.sync_copy(src_hbm.at[wid, 0], src_v.at[0])
    pltpu.sync_copy(zer_hbm, acc_sh.at[pl.ds(s * SL, SL)])
    plsc.subcore_barrier()

    pltpu.async_copy(tbl_hbm.at[src_v.at[0]], rows_v.at[0], sem)
    pltpu.sync_copy(src_hbm.at[wid, 1], src_v.at[1])

    def body(j, carry):
        buf = lax.rem(j, 2)
        nbuf = lax.rem(j + 1, 2)
        pltpu.make_async_copy(tbl_hbm.at[src_v.at[buf]], rows_v.at[buf],
                              sem).wait()

        @pl.when(j + 1 < NCHUNK)
        def _():
            pltpu.async_copy(tbl_hbm.at[src_v.at[nbuf]], rows_v.at[nbuf], sem)

        pltpu.sync_copy(rows_v.at[buf], acc_sh.at[dst_v.at[j]], add=True)

        @pl.when(j + 2 < NCHUNK)
        def _():
            pltpu.sync_copy(src_hbm.at[wid, j + 2], src_v.at[buf])
        return carry

    lax.fori_loop(0, NCHUNK, body, 0)
    plsc.subcore_barrier()
    pltpu.sync_copy(acc_sh.at[pl.ds(s * SL, SL)], out_hbm.at[c, pl.ds(s * SL, SL)])


def _tc_pre_body(x_ref, degp_ref, wpre_ref, bpre_ref, wg1_ref, o_ref):
    deg = degp_ref[0, :] + degp_ref[1, :] + 1.0
    dinv = lax.rsqrt(deg)[:, None][:N]
    h = jnp.dot(x_ref[...], wpre_ref[...],
                preferred_element_type=jnp.float32) + bpre_ref[...]
    hw = jnp.dot(h, wg1_ref[...], preferred_element_type=jnp.float32)
    o_ref[...] = hw * dinv


def _tc_mid_body(p_ref, hws_ref, degp_ref, bg_ref, wg2_ref, o_ref):
    deg = degp_ref[0, :] + degp_ref[1, :] + 1.0
    dinv = lax.rsqrt(deg)[:, None][:N]
    psum = p_ref[0, pl.ds(0, N), :] + p_ref[1, pl.ds(0, N), :]
    agg = dinv * (psum + hws_ref[...]) + bg_ref[...]
    h1 = jnp.maximum(agg, 0.0)
    o_ref[...] = jnp.dot(h1, wg2_ref[...],
                         preferred_element_type=jnp.float32) * dinv


def _tc_fin_body(p_ref, hws_ref, degp_ref, bg_ref, wpost_ref, bpost_ref,
                 ea_ref, wd0_ref, bd0_ref, wd1_ref, bd1_ref, wd2_ref,
                 bd2_ref, wd3_ref, bd3_ref, wfin_ref, bfin_ref, o_ref):
    deg = degp_ref[0, :] + degp_ref[1, :] + 1.0
    dinv = lax.rsqrt(deg)[:, None][:N]
    psum = p_ref[0, pl.ds(0, N), :] + p_ref[1, pl.ds(0, N), :]
    agg = dinv * (psum + hws_ref[...]) + bg_ref[...]
    h2 = jnp.maximum(agg, 0.0)
    feat = jnp.dot(h2, wpost_ref[...],
                   preferred_element_type=jnp.float32) + bpost_ref[...]
    e = jnp.maximum(jnp.dot(ea_ref[...], wd0_ref[...],
                            preferred_element_type=jnp.float32) + bd0_ref[...], 0.0)
    e = jnp.maximum(jnp.dot(e, wd1_ref[...],
                            preferred_element_type=jnp.float32) + bd1_ref[...], 0.0)
    e = jnp.maximum(jnp.dot(e, wd2_ref[...],
                            preferred_element_type=jnp.float32) + bd2_ref[...], 0.0)
    dist = jnp.dot(e, wd3_ref[...],
                   preferred_element_type=jnp.float32) + bd3_ref[...]
    merged = jnp.concatenate([feat, dist], axis=1)
    y = jnp.dot(merged, wfin_ref[...],
                preferred_element_type=jnp.float32) + bfin_ref[...]
    o_ref[...] = jax.nn.sigmoid(y)


def _tc_call(body, out_shape, *args):
    return pl.pallas_call(body, out_shape=out_shape)(*args)


def kernel(x, edge_index, edge_attr, W_pre, b_pre, Wg1, bg1, Wg2, bg2,
           W_post, b_post, Wd0, bd0, Wd1, bd1, Wd2, bd2, Wd3, bd3,
           W_fin, b_fin):
    f32 = jnp.float32
    src = edge_index[0]
    dst = edge_index[1]
    pad = EPAD - E
    pad_dst = N + jnp.arange(pad, dtype=jnp.int32) % (NP - N)
    src_p = jnp.concatenate([src, jnp.zeros((pad,), jnp.int32)]) \
               .reshape(NW, NCHUNK, CHUNK)
    dst_p = jnp.concatenate([dst, pad_dst]).reshape(NW, NCHUNK, CHUNK)
    K = edge_attr.shape[1]
    eap = jnp.pad(edge_attr, ((0, 0), (0, 8 - K)))
    wd0p = jnp.pad(Wd0, ((0, 8 - K), (0, 0)))
    zer_sl = jnp.zeros((SL, HID), f32)

    degp = _sc_degree(dst_p)

    hw1s = _tc_call(_tc_pre_body, jax.ShapeDtypeStruct((N, HID), f32),
                    x, degp, W_pre, b_pre, Wg1)
    p1 = _sc_layer(src_p, dst_p, hw1s, zer_sl)
    hw2s = _tc_call(_tc_mid_body, jax.ShapeDtypeStruct((N, HID), f32),
                    p1, hw1s, degp, bg1, Wg2)
    p2 = _sc_layer(src_p, dst_p, hw2s, zer_sl)

    y2d = _tc_call(_tc_fin_body, jax.ShapeDtypeStruct((N, 1), f32),
                   p2, hw2s, degp, bg2, W_post, b_post, eap, wd0p, bd0,
                   Wd1, bd1, Wd2, bd2, Wd3, bd3, W_fin, b_fin)
    return y2d[:, 0]


# speedup vs baseline: 1.1466x; 1.1466x over previous
import functools

import jax
import jax.numpy as jnp
from jax import lax
from jax.experimental import pallas as pl
from jax.experimental.pallas import tpu as pltpu
from jax.experimental.pallas import tpu_sc as plsc

N = 10000
NP = 10240
E = 320000
HID = 128
NCORES = 2
NSUB = 16
NW = NCORES * NSUB
CHUNK = 128
NCHUNK = 79
EPW = NCHUNK * CHUNK
EPAD = EPW * NW
SL = NP // NSUB

_mesh = plsc.VectorSubcoreMesh(core_axis_name="c", subcore_axis_name="s",
                               num_cores=NCORES, num_subcores=NSUB)


@functools.partial(
    pl.kernel,
    out_type=jax.ShapeDtypeStruct((NCORES, NP), jnp.float32),
    mesh=_mesh,
    scratch_types=[
        pltpu.VMEM((NCHUNK, CHUNK), jnp.int32),
        pltpu.VMEM((CHUNK,), jnp.float32),
        pltpu.VMEM((SL,), jnp.float32),
        pltpu.VMEM_SHARED((NP,), jnp.float32),
        pltpu.SemaphoreType.DMA,
    ],
)
def _sc_degree(dst_hbm, out_hbm, dst_v, ones_v, zer_v, acc_sh, sem):
    c = lax.axis_index("c")
    s = lax.axis_index("s")
    wid = c * NSUB + s
    pltpu.sync_copy(dst_hbm.at[wid], dst_v)
    for i in range(CHUNK // 16):
        ones_v[pl.ds(i * 16, 16)] = jnp.full((16,), 1.0, jnp.float32)
    for i in range(SL // 16):
        zer_v[pl.ds(i * 16, 16)] = jnp.zeros((16,), jnp.float32)
    pltpu.sync_copy(zer_v, acc_sh.at[pl.ds(s * SL, SL)])
    plsc.subcore_barrier()

    def body(j, carry):
        pltpu.sync_copy(ones_v, acc_sh.at[dst_v.at[j]], add=True)
        return carry

    lax.fori_loop(0, NCHUNK, body, 0)
    plsc.subcore_barrier()
    pltpu.sync_copy(acc_sh.at[pl.ds(s * SL, SL)], out_hbm.at[c, pl.ds(s * SL, SL)])


@functools.partial(
    pl.kernel,
    out_type=jax.ShapeDtypeStruct((NCORES, NP, HID), jnp.float32),
    mesh=_mesh,
    scratch_types=[
        pltpu.VMEM((2, CHUNK), jnp.int32),
        pltpu.VMEM((NCHUNK, CHUNK), jnp.int32),
        pltpu.VMEM((2, CHUNK, HID), jnp.float32),
        pltpu.VMEM_SHARED((NP, HID), jnp.float32),
        pltpu.SemaphoreType.DMA,
    ],
)
def _sc_layer(src_hbm, dst_hbm, tbl_hbm, zer_hbm, out_hbm,
              src_v, dst_v, rows_v, acc_sh, sem):
    c = lax.axis_index("c")
    s = lax.axis_index("s")
    wid = c * NSUB + s
    pltpu.sync_copy(dst_hbm.at[wid], dst_v)
    pltpu.sync_copy(src_hbm.at[wid, 0], src_v.at[0])
    pltpu.sync_copy(zer_hbm, acc_sh.at[pl.ds(s * SL, SL)])
    plsc.subcore_barrier()

    pltpu.async_copy(tbl_hbm.at[src_v.at[0]], rows_v.at[0], sem)
    pltpu.sync_copy(src_hbm.at[wid, 1], src_v.at[1])

    def body(j, carry):
        buf = lax.rem(j, 2)
        nbuf = lax.rem(j + 1, 2)
        pltpu.make_async_copy(tbl_hbm.at[src_v.at[buf]], rows_v.at[buf],
                              sem).wait()

        @pl.when(j + 1 < NCHUNK)
        def _():
            pltpu.async_copy(tbl_hbm.at[src_v.at[nbuf]], rows_v.at[nbuf], sem)

        pltpu.sync_copy(rows_v.at[buf], acc_sh.at[dst_v.at[j]], add=True)

        @pl.when(j + 2 < NCHUNK)
        def _():
            pltpu.sync_copy(src_hbm.at[wid, j + 2], src_v.at[buf])
        return carry

    lax.fori_loop(0, NCHUNK, body, 0)
    plsc.subcore_barrier()
    pltpu.sync_copy(acc_sh.at[pl.ds(s * SL, SL)], out_hbm.at[c, pl.ds(s * SL, SL)])


def _tc_pre_body(x_ref, degp_ref, wpre_ref, bpre_ref, wg1_ref, o_ref):
    deg = degp_ref[0, :] + degp_ref[1, :] + 1.0
    dinv = lax.rsqrt(deg)
    h = jnp.dot(x_ref[...], wpre_ref[...],
                preferred_element_type=jnp.float32) + bpre_ref[...]
    hw = jnp.dot(h, wg1_ref[...], preferred_element_type=jnp.float32)
    o_ref[...] = hw * dinv[:, None]


def _tc_mid_body(p_ref, hws_ref, degp_ref, bg_ref, wg2_ref, o_ref):
    deg = degp_ref[0, :] + degp_ref[1, :] + 1.0
    dinv = lax.rsqrt(deg)
    agg = dinv[:, None] * (p_ref[0] + p_ref[1] + hws_ref[...]) + bg_ref[...]
    h1 = jnp.maximum(agg, 0.0)
    o_ref[...] = jnp.dot(h1, wg2_ref[...],
                         preferred_element_type=jnp.float32) * dinv[:, None]


def _tc_fin_body(p_ref, hws_ref, degp_ref, bg_ref, wpost_ref, bpost_ref,
                 ea_ref, wd0_ref, bd0_ref, wd1_ref, bd1_ref, wd2_ref,
                 bd2_ref, wd3_ref, bd3_ref, wfin_ref, bfin_ref, o_ref):
    deg = degp_ref[0, :] + degp_ref[1, :] + 1.0
    dinv = lax.rsqrt(deg)
    agg = dinv[:, None] * (p_ref[0] + p_ref[1] + hws_ref[...]) + bg_ref[...]
    h2 = jnp.maximum(agg, 0.0)
    feat = jnp.dot(h2, wpost_ref[...],
                   preferred_element_type=jnp.float32) + bpost_ref[...]
    e = jnp.maximum(jnp.dot(ea_ref[...], wd0_ref[...],
                            preferred_element_type=jnp.float32) + bd0_ref[...], 0.0)
    e = jnp.maximum(jnp.dot(e, wd1_ref[...],
                            preferred_element_type=jnp.float32) + bd1_ref[...], 0.0)
    e = jnp.maximum(jnp.dot(e, wd2_ref[...],
                            preferred_element_type=jnp.float32) + bd2_ref[...], 0.0)
    dist = jnp.dot(e, wd3_ref[...],
                   preferred_element_type=jnp.float32) + bd3_ref[...]
    merged = jnp.concatenate([feat, dist], axis=1)
    y = jnp.dot(merged, wfin_ref[...],
                preferred_element_type=jnp.float32) + bfin_ref[...]
    o_ref[...] = jax.nn.sigmoid(y)


def _tc_call(body, out_shape, *args):
    return pl.pallas_call(body, out_shape=out_shape)(*args)


def kernel(x, edge_index, edge_attr, W_pre, b_pre, Wg1, bg1, Wg2, bg2,
           W_post, b_post, Wd0, bd0, Wd1, bd1, Wd2, bd2, Wd3, bd3,
           W_fin, b_fin):
    f32 = jnp.float32
    src = edge_index[0]
    dst = edge_index[1]
    pad = EPAD - E
    pad_dst = N + jnp.arange(pad, dtype=jnp.int32) % (NP - N)
    src_p = jnp.concatenate([src, jnp.zeros((pad,), jnp.int32)]) \
               .reshape(NW, NCHUNK, CHUNK)
    dst_p = jnp.concatenate([dst, pad_dst]).reshape(NW, NCHUNK, CHUNK)
    xp = jnp.concatenate([x, jnp.zeros((NP - N, x.shape[1]), f32)], axis=0)
    eap = jnp.zeros((NP, HID), f32).at[:N, :edge_attr.shape[1]].set(edge_attr)
    wd0p = jnp.zeros((HID, HID), f32).at[:Wd0.shape[0]].set(Wd0)
    zer_sl = jnp.zeros((SL, HID), f32)

    degp = _sc_degree(dst_p)

    hw1s = _tc_call(_tc_pre_body, jax.ShapeDtypeStruct((NP, HID), f32),
                    xp, degp, W_pre, b_pre, Wg1)
    p1 = _sc_layer(src_p, dst_p, hw1s, zer_sl)
    hw2s = _tc_call(_tc_mid_body, jax.ShapeDtypeStruct((NP, HID), f32),
                    p1, hw1s, degp, bg1, Wg2)
    p2 = _sc_layer(src_p, dst_p, hw2s, zer_sl)

    y2d = _tc_call(_tc_fin_body, jax.ShapeDtypeStruct((NP, 1), f32),
                   p2, hw2s, degp, bg2, W_post, b_post, eap, wd0p, bd0,
                   Wd1, bd1, Wd2, bd2, Wd3, bd3, W_fin, b_fin)
    return y2d[:N, 0]
